# Initial kernel scaffold; baseline (speedup 1.0000x reference)
import jax
import jax.numpy as jnp
from jax.experimental import pallas as pl


def kernel(x, edge_index, W1_0, b1_0, W2_0, b2_0, W1_1, b1_1, W2_1, b2_1, Wp, bp):
    raise NotImplementedError("write your pallas kernel here")


# speedup vs baseline: 3.4473x; 3.4473x over previous
import functools

import jax
import jax.numpy as jnp
from jax import lax
from jax.experimental import pallas as pl
from jax.experimental.pallas import tpu as pltpu
from jax.experimental.pallas import tpu_sc as plsc

N = 10000
E = 160000
D = 256
HALF = 128

NSC = 2
NSUB = 16
CHUNK = 128
NCHUNKS = -(-E // CHUNK)
NCHUNKS_PAD = -(-NCHUNKS // NSUB) * NSUB
NCH = NCHUNKS_PAD // NSUB
EP = NCHUNKS_PAD * CHUNK
NP = 10240
ROWS_PER_SUB = NP // NSUB
ZCH = ROWS_PER_SUB // CHUNK


def _sc_agg_body(hA, hB, srcp, dstp, zrows, aggA, aggB, sidx, didx, rows, aggS, sem):
    c = lax.axis_index("c")
    s = lax.axis_index("s")

    def run(h_hbm, out_hbm):
        pltpu.sync_copy(zrows, rows)

        def zbody(j, carry):
            r0 = pl.multiple_of((s * ZCH + j) * CHUNK, CHUNK)
            pltpu.sync_copy(rows, aggS.at[pl.ds(r0, CHUNK)])
            return carry

        lax.fori_loop(0, ZCH, zbody, 0)
        plsc.subcore_barrier()

        def ebody(i, carry):
            off = pl.multiple_of((s * NCH + i) * CHUNK, CHUNK)
            pltpu.sync_copy(srcp.at[pl.ds(off, CHUNK)], sidx)
            pltpu.sync_copy(dstp.at[pl.ds(off, CHUNK)], didx)
            pltpu.async_copy(h_hbm.at[sidx], rows, sem).wait()
            pltpu.sync_copy(rows, aggS.at[didx], add=True)
            return carry

        lax.fori_loop(0, NCH, ebody, 0)
        plsc.subcore_barrier()

        r0 = pl.multiple_of(s * ROWS_PER_SUB, ROWS_PER_SUB)
        pltpu.sync_copy(aggS.at[pl.ds(r0, ROWS_PER_SUB)],
                        out_hbm.at[pl.ds(r0, ROWS_PER_SUB)])

    @pl.when(c == 0)
    def _():
        run(hA, aggA)

    @pl.when(c == 1)
    def _():
        run(hB, aggB)


_sc_agg = functools.partial(
    pl.kernel,
    mesh=plsc.VectorSubcoreMesh(core_axis_name="c", subcore_axis_name="s"),
    out_type=[
        jax.ShapeDtypeStruct((NP, HALF), jnp.float32),
        jax.ShapeDtypeStruct((NP, HALF), jnp.float32),
    ],
    scratch_types=[
        pltpu.VMEM((CHUNK,), jnp.int32),
        pltpu.VMEM((CHUNK,), jnp.int32),
        pltpu.VMEM((CHUNK, HALF), jnp.float32),
        pltpu.VMEM_SHARED((NP, HALF), jnp.float32),
        pltpu.SemaphoreType.DMA,
    ],
)(_sc_agg_body)


def _mlp1_body(x_ref, aA_ref, aB_ref, W1_ref, b1_ref, W2_ref, b2_ref,
               oA_ref, oB_ref):
    agg = jnp.concatenate([aA_ref[...], aB_ref[...]], axis=1)
    m = x_ref[...] + agg
    t = jnp.dot(m, W1_ref[...], preferred_element_type=jnp.float32) + b1_ref[...]
    t = jnp.maximum(t, 0.0)
    h = jnp.dot(t, W2_ref[...], preferred_element_type=jnp.float32) + b2_ref[...]
    h = jnp.maximum(h, 0.0)
    oA_ref[...] = h[:, :HALF]
    oB_ref[...] = h[:, HALF:]


def _mlp2_body(hA_ref, hB_ref, aA_ref, aB_ref, W1_ref, b1_ref, W2_ref, b2_ref,
               Wp_ref, bp_ref, z_ref):
    m = jnp.concatenate([hA_ref[...] + aA_ref[...], hB_ref[...] + aB_ref[...]],
                        axis=1)
    t = jnp.dot(m, W1_ref[...], preferred_element_type=jnp.float32) + b1_ref[...]
    t = jnp.maximum(t, 0.0)
    h = jnp.dot(t, W2_ref[...], preferred_element_type=jnp.float32) + b2_ref[...]
    h = jnp.maximum(h, 0.0)
    z_ref[...] = jnp.dot(h, Wp_ref[...], preferred_element_type=jnp.float32) + bp_ref[...]


_RB = 1000
_GRID = N // _RB

_row_spec = pl.BlockSpec((_RB, D), lambda i: (i, 0))
_half_spec = pl.BlockSpec((_RB, HALF), lambda i: (i, 0))
_w_spec = pl.BlockSpec((D, D), lambda i: (0, 0))
_b_spec = pl.BlockSpec((1, D), lambda i: (0, 0))

_mlp1 = pl.pallas_call(
    _mlp1_body,
    grid=(_GRID,),
    in_specs=[_row_spec, _half_spec, _half_spec, _w_spec, _b_spec, _w_spec, _b_spec],
    out_specs=[_half_spec, _half_spec],
    out_shape=[
        jax.ShapeDtypeStruct((N, HALF), jnp.float32),
        jax.ShapeDtypeStruct((N, HALF), jnp.float32),
    ],
)

_mlp2 = pl.pallas_call(
    _mlp2_body,
    grid=(_GRID,),
    in_specs=[_half_spec, _half_spec, _half_spec, _half_spec,
              _w_spec, _b_spec, _w_spec, _b_spec, _w_spec, _b_spec],
    out_specs=_row_spec,
    out_shape=jax.ShapeDtypeStruct((N, D), jnp.float32),
)


def kernel(x, edge_index, W1_0, b1_0, W2_0, b2_0, W1_1, b1_1, W2_1, b2_1, Wp, bp):
    src = edge_index[0]
    dst = edge_index[1]
    pad = EP - E
    srcp = jnp.concatenate([src, jnp.zeros((pad,), jnp.int32)])
    dstp = jnp.concatenate([dst, jnp.full((pad,), NP - 1, jnp.int32)])
    zrows = jnp.zeros((CHUNK, HALF), jnp.float32)

    xA = x[:, :HALF]
    xB = x[:, HALF:]
    b1_0r = b1_0.reshape(1, D)
    b2_0r = b2_0.reshape(1, D)
    b1_1r = b1_1.reshape(1, D)
    b2_1r = b2_1.reshape(1, D)
    bpr = bp.reshape(1, D)

    a1A, a1B = _sc_agg(xA, xB, srcp, dstp, zrows)
    h1A, h1B = _mlp1(x, a1A, a1B, W1_0, b1_0r, W2_0, b2_0r)
    a2A, a2B = _sc_agg(h1A, h1B, srcp, dstp, zrows)
    z = _mlp2(h1A, h1B, a2A, a2B, W1_1, b1_1r, W2_1, b2_1r, Wp, bpr)
    return z
